# baseline (speedup 1.0000x reference)
import functools

import jax
import jax.numpy as jnp
from jax import lax
from jax.experimental import pallas as pl
from jax.experimental.pallas import tpu as pltpu
from jax.experimental.pallas import tpu_sc as plsc

MAX_NUM = 100000
HIDDEN = 256
EMBED = 128
BATCH = 16384
FIELDS = 26
TOTAL = BATCH * FIELDS

NW = 32
ROWS_PER_W = TOTAL // NW
CHUNK = 128
NCHUNK = ROWS_PER_W // CHUNK

PROJ_BLOCK = 4000


def _proj_body(enc_ref, w_ref, b_ref, out_ref):
    out_ref[...] = (
        jnp.dot(enc_ref[...], w_ref[...], preferred_element_type=jnp.float32)
        + b_ref[...]
    )


def _project_table(encodings, W, b2):
    return pl.pallas_call(
        _proj_body,
        grid=(MAX_NUM // PROJ_BLOCK,),
        in_specs=[
            pl.BlockSpec((PROJ_BLOCK, HIDDEN), lambda i: (i, 0)),
            pl.BlockSpec((HIDDEN, EMBED), lambda i: (0, 0)),
            pl.BlockSpec((1, EMBED), lambda i: (0, 0)),
        ],
        out_specs=pl.BlockSpec((PROJ_BLOCK, EMBED), lambda i: (i, 0)),
        out_shape=jax.ShapeDtypeStruct((MAX_NUM, EMBED), jnp.float32),
    )(encodings, W, b2)


STEP = 256
NSTEP = ROWS_PER_W // STEP


@functools.cache
def _make_gather():
    mesh = plsc.VectorSubcoreMesh(core_axis_name="c", subcore_axis_name="s")

    @functools.partial(
        pl.kernel,
        mesh=mesh,
        out_type=jax.ShapeDtypeStruct((TOTAL, EMBED), jnp.float32),
        scratch_types=[
            pltpu.VMEM((NCHUNK, CHUNK), jnp.int32),
            pltpu.VMEM((STEP, EMBED), jnp.float32),
            pltpu.VMEM((STEP, EMBED), jnp.float32),
            pltpu.SemaphoreType.DMA,
            pltpu.SemaphoreType.DMA,
            pltpu.SemaphoreType.DMA,
            pltpu.SemaphoreType.DMA,
        ],
    )
    def _gather_rows(table_hbm, idx_hbm, out_hbm, idx_v, buf_a, buf_b,
                     gs_a, gs_b, ws_a, ws_b):
        wid = lax.axis_index("s") * 2 + lax.axis_index("c")
        base = wid * ROWS_PER_W
        pltpu.sync_copy(idx_hbm.at[pl.ds(wid * NCHUNK, NCHUNK)], idx_v)

        def start_gather(s, buf, sem):
            c0 = pltpu.async_copy(table_hbm.at[idx_v.at[2 * s]],
                                  buf.at[pl.ds(0, CHUNK)], sem)
            c1 = pltpu.async_copy(table_hbm.at[idx_v.at[2 * s + 1]],
                                  buf.at[pl.ds(CHUNK, CHUNK)], sem)
            return c0, c1

        start_gather(0, buf_a, gs_a)

        def body(i, carry):
            s0 = 2 * i
            s1 = 2 * i + 1

            @pl.when(s1 < NSTEP)
            def _():
                start_gather(s1, buf_b, gs_b)

            c0, c1 = _wait_descs(table_hbm, idx_v, s0, buf_a, gs_a)
            wb_a = pltpu.async_copy(
                buf_a, out_hbm.at[pl.ds(base + s0 * STEP, STEP)], ws_a)
            wb_a.wait()

            @pl.when(s0 + 2 < NSTEP)
            def _():
                start_gather(s0 + 2, buf_a, gs_a)

            @pl.when(s1 < NSTEP)
            def _():
                _wait_descs(table_hbm, idx_v, s1, buf_b, gs_b)
                wb_b = pltpu.async_copy(
                    buf_b, out_hbm.at[pl.ds(base + s1 * STEP, STEP)], ws_b)
                wb_b.wait()

            return carry

        lax.fori_loop(0, (NSTEP + 1) // 2, body, 0)

    return _gather_rows


def _wait_descs(table_hbm, idx_v, s, buf, sem):
    c0 = pltpu.make_async_copy(table_hbm.at[idx_v.at[2 * s]],
                               buf.at[pl.ds(0, CHUNK)], sem)
    c1 = pltpu.make_async_copy(table_hbm.at[idx_v.at[2 * s + 1]],
                               buf.at[pl.ds(CHUNK, CHUNK)], sem)
    c0.wait()
    c1.wait()
    return c0, c1


J = 1000
M = MAX_NUM // J


def _proj_rot_body(s_ref, c_ref, we_ref, wo_ref, cm_ref, sm_ref, b_ref,
                   out_ref):
    cm = cm_ref[0]
    sm = sm_ref[0]
    a_mat = cm * we_ref[...] - sm * wo_ref[...]
    b_mat = sm * we_ref[...] + cm * wo_ref[...]
    out_ref[...] = (
        jnp.dot(s_ref[...], a_mat, preferred_element_type=jnp.float32)
        + jnp.dot(c_ref[...], b_mat, preferred_element_type=jnp.float32)
        + b_ref[...]
    )


def _project_table_rot(encodings, W, b2):
    S = encodings[:J, 0::2]
    C = encodings[:J, 1::2]
    sinm = encodings[::J, 0::2][..., None]
    cosm = encodings[::J, 1::2][..., None]
    We = W[0::2]
    Wo = W[1::2]
    return pl.pallas_call(
        _proj_rot_body,
        grid=(M,),
        in_specs=[
            pl.BlockSpec((J, EMBED), lambda m: (0, 0)),
            pl.BlockSpec((J, EMBED), lambda m: (0, 0)),
            pl.BlockSpec((EMBED, EMBED), lambda m: (0, 0)),
            pl.BlockSpec((EMBED, EMBED), lambda m: (0, 0)),
            pl.BlockSpec((1, EMBED, 1), lambda m: (m, 0, 0)),
            pl.BlockSpec((1, EMBED, 1), lambda m: (m, 0, 0)),
            pl.BlockSpec((1, EMBED), lambda m: (0, 0)),
        ],
        out_specs=pl.BlockSpec((J, EMBED), lambda m: (m, 0)),
        out_shape=jax.ShapeDtypeStruct((MAX_NUM, EMBED), jnp.float32),
    )(S, C, We, Wo, cosm, sinm, b2)


def kernel(num, encodings, W, b):
    table = _project_table_rot(encodings, W, b.reshape(1, EMBED))
    idx = num.astype(jnp.int32).T.reshape(TOTAL // CHUNK, CHUNK)
    flat = _make_gather()(table, idx)
    return flat.reshape(FIELDS, BATCH, EMBED).transpose(1, 0, 2)


# speedup vs baseline: 1.6540x; 1.6540x over previous
import functools

import jax
import jax.numpy as jnp
from jax import lax
from jax.experimental import pallas as pl
from jax.experimental.pallas import tpu as pltpu
from jax.experimental.pallas import tpu_sc as plsc

MAX_NUM = 100000
HIDDEN = 256
EMBED = 128
BATCH = 16384
FIELDS = 26
TOTAL = BATCH * FIELDS

NW = 32
ROWS_PER_W = TOTAL // NW
CHUNK = 128
NCHUNK = ROWS_PER_W // CHUNK

PROJ_BLOCK = 4000


def _proj_body(enc_ref, w_ref, b_ref, out_ref):
    out_ref[...] = (
        jnp.dot(enc_ref[...], w_ref[...], preferred_element_type=jnp.float32)
        + b_ref[...]
    )


def _project_table(encodings, W, b2):
    return pl.pallas_call(
        _proj_body,
        grid=(MAX_NUM // PROJ_BLOCK,),
        in_specs=[
            pl.BlockSpec((PROJ_BLOCK, HIDDEN), lambda i: (i, 0)),
            pl.BlockSpec((HIDDEN, EMBED), lambda i: (0, 0)),
            pl.BlockSpec((1, EMBED), lambda i: (0, 0)),
        ],
        out_specs=pl.BlockSpec((PROJ_BLOCK, EMBED), lambda i: (i, 0)),
        out_shape=jax.ShapeDtypeStruct((MAX_NUM, EMBED), jnp.float32),
    )(encodings, W, b2)


STEP = 256
NSTEP = ROWS_PER_W // STEP


@functools.cache
def _make_gather():
    mesh = plsc.VectorSubcoreMesh(core_axis_name="c", subcore_axis_name="s")

    @functools.partial(
        pl.kernel,
        mesh=mesh,
        out_type=jax.ShapeDtypeStruct((TOTAL, EMBED), jnp.float32),
        scratch_types=[
            pltpu.VMEM((NCHUNK, CHUNK), jnp.int32),
            pltpu.VMEM((STEP, EMBED), jnp.float32),
            pltpu.VMEM((STEP, EMBED), jnp.float32),
            pltpu.SemaphoreType.DMA,
            pltpu.SemaphoreType.DMA,
            pltpu.SemaphoreType.DMA,
            pltpu.SemaphoreType.DMA,
        ],
    )
    def _gather_rows(table_hbm, idx_hbm, out_hbm, idx_v, buf_a, buf_b,
                     gs_a, gs_b, ws_a, ws_b):
        wid = lax.axis_index("s") * 2 + lax.axis_index("c")
        base = wid * ROWS_PER_W
        pltpu.sync_copy(idx_hbm.at[pl.ds(wid * NCHUNK, NCHUNK)], idx_v)

        def start_gather(s, buf, sem):
            c0 = pltpu.async_copy(table_hbm.at[idx_v.at[2 * s]],
                                  buf.at[pl.ds(0, CHUNK)], sem)
            c1 = pltpu.async_copy(table_hbm.at[idx_v.at[2 * s + 1]],
                                  buf.at[pl.ds(CHUNK, CHUNK)], sem)
            return c0, c1

        start_gather(0, buf_a, gs_a)

        def body(i, carry):
            s0 = 2 * i
            s1 = 2 * i + 1

            @pl.when(s1 < NSTEP)
            def _():
                start_gather(s1, buf_b, gs_b)

            c0, c1 = _wait_descs(table_hbm, idx_v, s0, buf_a, gs_a)
            wb_a = pltpu.async_copy(
                buf_a, out_hbm.at[pl.ds(base + s0 * STEP, STEP)], ws_a)
            wb_a.wait()

            @pl.when(s0 + 2 < NSTEP)
            def _():
                start_gather(s0 + 2, buf_a, gs_a)

            @pl.when(s1 < NSTEP)
            def _():
                _wait_descs(table_hbm, idx_v, s1, buf_b, gs_b)
                wb_b = pltpu.async_copy(
                    buf_b, out_hbm.at[pl.ds(base + s1 * STEP, STEP)], ws_b)
                wb_b.wait()

            return carry

        lax.fori_loop(0, (NSTEP + 1) // 2, body, 0)

    return _gather_rows


def _wait_descs(table_hbm, idx_v, s, buf, sem):
    c0 = pltpu.make_async_copy(table_hbm.at[idx_v.at[2 * s]],
                               buf.at[pl.ds(0, CHUNK)], sem)
    c1 = pltpu.make_async_copy(table_hbm.at[idx_v.at[2 * s + 1]],
                               buf.at[pl.ds(CHUNK, CHUNK)], sem)
    c0.wait()
    c1.wait()
    return c0, c1


def kernel(num, encodings, W, b):
    table = _project_table(encodings, W, b.reshape(1, EMBED))
    idx = num.astype(jnp.int32).T.reshape(TOTAL // CHUNK, CHUNK)
    flat = _make_gather()(table, idx)
    return flat.reshape(FIELDS, BATCH, EMBED).transpose(1, 0, 2)
